# baseline (speedup 1.0000x reference)
import jax
import jax.numpy as jnp
from jax.experimental import pallas as pl
from jax.experimental.pallas import tpu as pltpu

_NSPLIT = 8


def _body(cache_ref, update_ref, out_ref, cache_sems, upd_sem):
    B = cache_ref.shape[0]
    S = cache_ref.shape[1]
    U = update_ref.shape[1]
    bs = B // _NSPLIT
    copies = []
    for i in range(_NSPLIT):
        copies.append(
            pltpu.make_async_copy(
                cache_ref.at[pl.ds(i * bs, bs), pl.ds(U, S - U), :],
                out_ref.at[pl.ds(i * bs, bs), pl.ds(0, S - U), :],
                cache_sems.at[i],
            )
        )
    upd = pltpu.make_async_copy(
        update_ref, out_ref.at[:, pl.ds(S - U, U), :], upd_sem
    )
    for c in copies:
        c.start()
    upd.start()
    for c in copies:
        c.wait()
    upd.wait()


def kernel(cache, update):
    return pl.pallas_call(
        _body,
        in_specs=[
            pl.BlockSpec(memory_space=pl.MemorySpace.ANY),
            pl.BlockSpec(memory_space=pl.MemorySpace.ANY),
        ],
        out_specs=pl.BlockSpec(memory_space=pl.MemorySpace.ANY),
        out_shape=jax.ShapeDtypeStruct(cache.shape, cache.dtype),
        scratch_shapes=[
            pltpu.SemaphoreType.DMA((_NSPLIT,)),
            pltpu.SemaphoreType.DMA,
        ],
    )(cache, update)


# speedup vs baseline: 48.4015x; 48.4015x over previous
import jax
import jax.numpy as jnp
from jax.experimental import pallas as pl
from jax.experimental.pallas import tpu as pltpu


def _body(cache_ref, update_ref, out_ref):
    S = cache_ref.shape[1]
    U = update_ref.shape[1]
    out_ref[:, pl.ds(0, S - U), :] = cache_ref[:, pl.ds(U, S - U), :]
    out_ref[:, pl.ds(S - U, U), :] = update_ref[...]


def kernel(cache, update):
    B, S, D = cache.shape
    U = update.shape[1]
    bb = 1
    return pl.pallas_call(
        _body,
        grid=(B // bb,),
        in_specs=[
            pl.BlockSpec((bb, S, D), lambda i: (i, 0, 0)),
            pl.BlockSpec((bb, U, D), lambda i: (i, 0, 0)),
        ],
        out_specs=pl.BlockSpec((bb, S, D), lambda i: (i, 0, 0)),
        out_shape=jax.ShapeDtypeStruct(cache.shape, cache.dtype),
    )(cache, update)
